# baseline (speedup 1.0000x reference)
import functools

import jax
import jax.numpy as jnp
from jax import lax
from jax.experimental import pallas as pl
from jax.experimental.pallas import tpu as pltpu
from jax.experimental.pallas import tpu_sc as plsc

NUM_EMB = 1000000
DIM = 64
BATCH = 16384
FIELDS = 26
NW = 32
B_PER_W = BATCH // NW
N_BLOCK = FIELDS
NBUF = 2


def _make_gather():
    mesh = plsc.VectorSubcoreMesh(core_axis_name="c", subcore_axis_name="s")

    @functools.partial(
        pl.kernel,
        mesh=mesh,
        out_type=jax.ShapeDtypeStruct((FIELDS, BATCH, DIM), jnp.float32),
        scratch_types=[
            pltpu.VMEM((FIELDS, B_PER_W), jnp.int32),
            pltpu.VMEM((NBUF, B_PER_W, DIM), jnp.float32),
            pltpu.SemaphoreType.DMA,
            pltpu.SemaphoreType.DMA,
            pltpu.SemaphoreType.DMA,
            pltpu.SemaphoreType.DMA,
        ],
        compiler_params=pltpu.CompilerParams(use_tc_tiling_on_sc=False),
    )
    def gather_kernel(tok_hbm, table_hbm, out_hbm, idx_v, rows_v,
                      gs0, gs1, os0, os1):
        gsems = [gs0, gs1]
        osems = [os0, os1]
        wid = lax.axis_index("s") * 2 + lax.axis_index("c")
        b_base = wid * B_PER_W

        pltpu.sync_copy(tok_hbm.at[:, pl.ds(b_base, B_PER_W)], idx_v)

        def fire_gather(f, b):
            pltpu.async_copy(
                table_hbm.at[idx_v.at[f]], rows_v.at[b], gsems[b]
            )

        def drain_gather(b):
            pltpu.make_async_copy(
                table_hbm.at[pl.ds(0, B_PER_W)], rows_v.at[b], gsems[b]
            ).wait()

        def fire_store(f, b):
            pltpu.async_copy(
                rows_v.at[b],
                out_hbm.at[f, pl.ds(b_base, B_PER_W)],
                osems[b],
            )

        def drain_store(b):
            pltpu.make_async_copy(
                rows_v.at[b], out_hbm.at[0, pl.ds(0, B_PER_W)], osems[b]
            ).wait()

        fire_gather(0, 0)
        for f in range(N_BLOCK):
            nxt = f + 1
            if nxt < N_BLOCK:
                b_nxt = nxt % NBUF
                if nxt >= NBUF:
                    drain_store(b_nxt)
                fire_gather(nxt, b_nxt)
            b = f % NBUF
            drain_gather(b)
            fire_store(f, b)
        for f in range(N_BLOCK - NBUF, N_BLOCK):
            drain_store(f % NBUF)

    return gather_kernel


_gather = _make_gather()


def _tc_transpose_body(x_ref, o_ref):
    x = x_ref[0]
    ys = []
    for j in range(8):
        z = x[j].reshape(128, DIM).T
        ys.append(z.reshape(8, 8, 128))
    o_ref[0] = jnp.stack(ys, axis=1)


_tc_transpose = pl.pallas_call(
    _tc_transpose_body,
    grid=(FIELDS, BATCH // 128 // 8),
    in_specs=[
        pl.BlockSpec((1, 8, DIM, 128), lambda f, t: (f, t, 0, 0)),
    ],
    out_specs=pl.BlockSpec((1, 8, 8, 8, 128), lambda f, t: (f, 0, t, 0, 0)),
    out_shape=jax.ShapeDtypeStruct((FIELDS, 8, BATCH // 128, 8, 128),
                                   jnp.float32),
)


def kernel(tokens_ids, weights):
    tok_t = tokens_ids.astype(jnp.int32).T
    out3 = _gather(tok_t, weights)
    in4 = out3.reshape(FIELDS, BATCH // 128, DIM, 128)
    k5 = _tc_transpose(in4)
    return k5.transpose(2, 4, 0, 1, 3).reshape(BATCH, FIELDS, DIM)


# speedup vs baseline: 1.1812x; 1.1812x over previous
import functools

import jax
import jax.numpy as jnp
from jax import lax
from jax.experimental import pallas as pl
from jax.experimental.pallas import tpu as pltpu
from jax.experimental.pallas import tpu_sc as plsc

NUM_EMB = 1000000
DIM = 64
BATCH = 16384
FIELDS = 26
NW = 32
B_PER_W = BATCH // NW
N_BLOCK = FIELDS
NBUF = 2


def _make_gather():
    mesh = plsc.VectorSubcoreMesh(core_axis_name="c", subcore_axis_name="s")

    @functools.partial(
        pl.kernel,
        mesh=mesh,
        out_type=jax.ShapeDtypeStruct((FIELDS, BATCH, DIM), jnp.float32),
        scratch_types=[
            pltpu.VMEM((FIELDS, B_PER_W), jnp.int32),
            pltpu.VMEM((NBUF, B_PER_W, DIM), jnp.float32),
            pltpu.SemaphoreType.DMA,
            pltpu.SemaphoreType.DMA,
            pltpu.SemaphoreType.DMA,
            pltpu.SemaphoreType.DMA,
        ],
        compiler_params=pltpu.CompilerParams(use_tc_tiling_on_sc=False),
    )
    def gather_kernel(tok_hbm, table_hbm, out_hbm, idx_v, rows_v,
                      gs0, gs1, os0, os1):
        gsems = [gs0, gs1]
        osems = [os0, os1]
        wid = lax.axis_index("s") * 2 + lax.axis_index("c")
        b_base = wid * B_PER_W

        pltpu.sync_copy(tok_hbm.at[:, pl.ds(b_base, B_PER_W)], idx_v)

        def fire_gather(f, b):
            pltpu.async_copy(
                table_hbm.at[idx_v.at[f]], rows_v.at[b], gsems[b]
            )

        def drain_gather(b):
            pltpu.make_async_copy(
                table_hbm.at[pl.ds(0, B_PER_W)], rows_v.at[b], gsems[b]
            ).wait()

        def fire_store(f, b):
            pltpu.async_copy(
                rows_v.at[b],
                out_hbm.at[f, pl.ds(b_base, B_PER_W)],
                osems[b],
            )

        def drain_store(b):
            pltpu.make_async_copy(
                rows_v.at[b], out_hbm.at[0, pl.ds(0, B_PER_W)], osems[b]
            ).wait()

        fire_gather(0, 0)
        for f in range(N_BLOCK):
            nxt = f + 1
            if nxt < N_BLOCK:
                b_nxt = nxt % NBUF
                if nxt >= NBUF:
                    drain_store(b_nxt)
                fire_gather(nxt, b_nxt)
            b = f % NBUF
            drain_gather(b)
            fire_store(f, b)
        for f in range(N_BLOCK - NBUF, N_BLOCK):
            drain_store(f % NBUF)

    return gather_kernel


_gather = _make_gather()


def _tc_transpose_body(x_ref, o_ref):
    x = x_ref[0]
    ys = []
    for j in range(8):
        z = x[j].reshape(128, DIM).T
        ys.append(z.reshape(8, 8, 128))
    o_ref[0] = jnp.stack(ys, axis=1)


_tc_transpose = pl.pallas_call(
    _tc_transpose_body,
    grid=(FIELDS, BATCH // 128 // 8),
    in_specs=[
        pl.BlockSpec((1, 8, DIM, 128), lambda f, t: (f, t, 0, 0)),
    ],
    out_specs=pl.BlockSpec((1, 8, 8, 8, 128), lambda f, t: (f, 0, t, 0, 0)),
    out_shape=jax.ShapeDtypeStruct((FIELDS, 8, BATCH // 128, 8, 128),
                                   jnp.float32),
)


def kernel(tokens_ids, weights):
    tok_t = tokens_ids.astype(jnp.int32).T
    out3 = _gather(tok_t, weights)
    return out3.transpose(1, 0, 2)
